# Initial kernel scaffold; baseline (speedup 1.0000x reference)
import jax
import jax.numpy as jnp
from jax.experimental import pallas as pl


def kernel(x, edge_index, W, b):
    raise NotImplementedError("write your pallas kernel here")


# speedup vs baseline: 2.9656x; 2.9656x over previous
import functools

import jax
import jax.numpy as jnp
from jax import lax
from jax.experimental import pallas as pl
from jax.experimental.pallas import tpu as pltpu
from jax.experimental.pallas import tpu_sc as plsc

N_NODES = 10000
D = 128
NC = 2
NS = 16
NW = NC * NS
CH = 128
N_PAD = 10112
ZROWS = N_PAD // NS


def _sc_segment_sum(x, src2d, dst2d, zeros, cpw):
    mesh = plsc.VectorSubcoreMesh(core_axis_name="c", subcore_axis_name="s")

    @functools.partial(
        pl.kernel,
        out_type=jax.ShapeDtypeStruct((NC * N_PAD, D), jnp.float32),
        mesh=mesh,
        scratch_types=[
            pltpu.VMEM_SHARED((N_PAD, D), jnp.float32),
            pltpu.VMEM((cpw, CH), jnp.int32),
            pltpu.VMEM((cpw, CH), jnp.int32),
            pltpu.VMEM((CH, D), jnp.float32),
            pltpu.SemaphoreType.DMA,
        ],
    )
    def seg_sum(x_hbm, src_hbm, dst_hbm, z_hbm, out_hbm, acc, sidx, didx, rows, sem):
        c = lax.axis_index("c")
        s = lax.axis_index("s")
        wid = s * NC + c

        pltpu.sync_copy(z_hbm, rows)
        zbase = s * ZROWS
        off = 0
        while off < ZROWS:
            nr = min(CH, ZROWS - off)
            pltpu.sync_copy(rows.at[pl.ds(0, nr)], acc.at[pl.ds(zbase + off, nr)])
            off += nr
        plsc.subcore_barrier()

        pltpu.sync_copy(src_hbm.at[pl.ds(wid * cpw, cpw)], sidx)
        pltpu.sync_copy(dst_hbm.at[pl.ds(wid * cpw, cpw)], didx)

        def body(j, carry):
            pltpu.async_copy(x_hbm.at[sidx.at[j]], rows, sem).wait()
            pltpu.sync_copy(rows, acc.at[didx.at[j]], add=True)
            return carry

        lax.fori_loop(0, cpw, body, 0)
        plsc.subcore_barrier()

        obase = s * ZROWS
        off = 0
        while off < ZROWS:
            nr = min(CH, ZROWS - off)
            r0 = obase + off
            pltpu.sync_copy(acc.at[pl.ds(r0, nr)], rows.at[pl.ds(0, nr)])
            pltpu.sync_copy(rows.at[pl.ds(0, nr)],
                            out_hbm.at[pl.ds(c * N_PAD + r0, nr)])
            off += nr

    return seg_sum(x, src2d, dst2d, zeros)


def _tc_body(x_ref, p0_ref, p1_ref, wt_ref, b_ref, out_ref):
    m = p0_ref[...] + p1_ref[...]
    a = jnp.dot(x_ref[...], wt_ref[0:D, :], preferred_element_type=jnp.float32)
    a = a + jnp.dot(m, wt_ref[D:2 * D, :], preferred_element_type=jnp.float32)
    out_ref[...] = jnp.maximum(a + b_ref[...], 0.0)


def kernel(x, edge_index, W, b):
    n, d = x.shape
    e = edge_index.shape[1]
    assert n == N_NODES and d == D

    cpw = -(-e // (NW * CH))
    cpw = (cpw + 7) // 8 * 8
    e_pad = NW * CH * cpw
    src = edge_index[0].astype(jnp.int32)
    dst = edge_index[1].astype(jnp.int32)
    pad = e_pad - e
    src2d = jnp.concatenate([src, jnp.zeros((pad,), jnp.int32)]).reshape(-1, CH)
    dst2d = jnp.concatenate([dst, jnp.full((pad,), N_NODES, jnp.int32)]).reshape(-1, CH)
    zeros = jnp.zeros((CH, D), jnp.float32)

    partials = _sc_segment_sum(x, src2d, dst2d, zeros, cpw)
    p0 = partials[:N_NODES]
    p1 = partials[N_PAD:N_PAD + N_NODES]

    wt = W.T
    b2 = b.reshape(1, D)
    br = 1000
    out = pl.pallas_call(
        _tc_body,
        grid=(N_NODES // br,),
        in_specs=[
            pl.BlockSpec((br, D), lambda i: (i, 0)),
            pl.BlockSpec((br, D), lambda i: (i, 0)),
            pl.BlockSpec((br, D), lambda i: (i, 0)),
            pl.BlockSpec((2 * D, D), lambda i: (0, 0)),
            pl.BlockSpec((1, D), lambda i: (0, 0)),
        ],
        out_specs=pl.BlockSpec((br, D), lambda i: (i, 0)),
        out_shape=jax.ShapeDtypeStruct((N_NODES, D), jnp.float32),
    )(x, p0, p1, wt, b2)
    return out


# speedup vs baseline: 3.1593x; 1.0653x over previous
import functools

import jax
import jax.numpy as jnp
from jax import lax
from jax.experimental import pallas as pl
from jax.experimental.pallas import tpu as pltpu
from jax.experimental.pallas import tpu_sc as plsc

N_NODES = 10000
D = 128
NC = 2
NS = 16
NW = NC * NS
CH = 128
N_PAD = 10112
ZROWS = N_PAD // NS


def _sc_segment_sum(x, src2d, dst2d, zeros, cpw):
    mesh = plsc.VectorSubcoreMesh(core_axis_name="c", subcore_axis_name="s")

    assert cpw % 4 == 0 and cpw >= 8
    L = cpw // 2

    @functools.partial(
        pl.kernel,
        out_type=jax.ShapeDtypeStruct((NC * N_PAD, D), jnp.float32),
        mesh=mesh,
        scratch_types=[
            pltpu.VMEM_SHARED((N_PAD, D), jnp.float32),
            pltpu.VMEM((L, CH), jnp.int32),
            pltpu.VMEM((L, CH), jnp.int32),
            pltpu.VMEM((2, CH, D), jnp.float32),
            pltpu.SemaphoreType.DMA,
            pltpu.SemaphoreType.DMA,
        ],
    )
    def seg_sum(x_hbm, src_hbm, dst_hbm, z_hbm, out_hbm, acc, sidx, didx, bufs,
                g0, g1):
        gs = (g0, g1)
        c = lax.axis_index("c")
        s = lax.axis_index("s")
        wid = s * NC + c

        pltpu.sync_copy(z_hbm, bufs.at[0])
        zbase = s * ZROWS
        off = 0
        while off < ZROWS:
            nr = min(CH, ZROWS - off)
            pltpu.sync_copy(bufs.at[0].at[pl.ds(0, nr)],
                            acc.at[pl.ds(zbase + off, nr)])
            off += nr
        plsc.subcore_barrier()

        def start_gather(j, b):
            pltpu.async_copy(x_hbm.at[sidx.at[j]], bufs.at[b], gs[b])

        def wait_gather(j, b):
            pltpu.make_async_copy(x_hbm.at[sidx.at[j]], bufs.at[b], gs[b]).wait()

        for p in range(2):
            pltpu.sync_copy(src_hbm.at[pl.ds(wid * cpw + p * L, L)], sidx)
            pltpu.sync_copy(dst_hbm.at[pl.ds(wid * cpw + p * L, L)], didx)
            start_gather(0, 0)

            def slot(j, b):
                wait_gather(j, b)
                start_gather(lax.rem(j + 1, L), 1 - b)
                pltpu.sync_copy(bufs.at[b], acc.at[didx.at[j]], add=True)

            def round_body(g, carry):
                slot(2 * g, 0)
                slot(2 * g + 1, 1)
                return carry

            lax.fori_loop(0, L // 2, round_body, 0)
            wait_gather(0, 0)
        plsc.subcore_barrier()

        obase = s * ZROWS
        off = 0
        while off < ZROWS:
            nr = min(CH, ZROWS - off)
            r0 = obase + off
            pltpu.sync_copy(acc.at[pl.ds(r0, nr)], bufs.at[0].at[pl.ds(0, nr)])
            pltpu.sync_copy(bufs.at[0].at[pl.ds(0, nr)],
                            out_hbm.at[pl.ds(c * N_PAD + r0, nr)])
            off += nr

    return seg_sum(x, src2d, dst2d, zeros)


def _tc_body(x_ref, p0_ref, p1_ref, wt_ref, b_ref, out_ref):
    m = p0_ref[...] + p1_ref[...]
    a = jnp.dot(x_ref[...], wt_ref[0:D, :], preferred_element_type=jnp.float32)
    a = a + jnp.dot(m, wt_ref[D:2 * D, :], preferred_element_type=jnp.float32)
    out_ref[...] = jnp.maximum(a + b_ref[...], 0.0)


def kernel(x, edge_index, W, b):
    n, d = x.shape
    e = edge_index.shape[1]
    assert n == N_NODES and d == D

    cpw = -(-e // (NW * CH))
    cpw = (cpw + 7) // 8 * 8
    e_pad = NW * CH * cpw
    src = edge_index[0].astype(jnp.int32)
    dst = edge_index[1].astype(jnp.int32)
    pad = e_pad - e
    src2d = jnp.concatenate([src, jnp.zeros((pad,), jnp.int32)]).reshape(-1, CH)
    dst2d = jnp.concatenate([dst, jnp.full((pad,), N_NODES, jnp.int32)]).reshape(-1, CH)
    zeros = jnp.zeros((CH, D), jnp.float32)

    partials = _sc_segment_sum(x, src2d, dst2d, zeros, cpw)
    p0 = partials[:N_NODES]
    p1 = partials[N_PAD:N_PAD + N_NODES]

    wt = W.T
    b2 = b.reshape(1, D)
    br = 1000
    out = pl.pallas_call(
        _tc_body,
        grid=(N_NODES // br,),
        in_specs=[
            pl.BlockSpec((br, D), lambda i: (i, 0)),
            pl.BlockSpec((br, D), lambda i: (i, 0)),
            pl.BlockSpec((br, D), lambda i: (i, 0)),
            pl.BlockSpec((2 * D, D), lambda i: (0, 0)),
            pl.BlockSpec((1, D), lambda i: (0, 0)),
        ],
        out_specs=pl.BlockSpec((br, D), lambda i: (i, 0)),
        out_shape=jax.ShapeDtypeStruct((N_NODES, D), jnp.float32),
    )(x, p0, p1, wt, b2)
    return out


# speedup vs baseline: 10.2692x; 3.2505x over previous
import functools

import jax
import jax.numpy as jnp
from jax import lax
from jax.experimental import pallas as pl
from jax.experimental.pallas import tpu as pltpu
from jax.experimental.pallas import tpu_sc as plsc

N_NODES = 10000
D = 128
NC = 2
NS = 16
NW = NC * NS
CH = 125
OCH = 80
NCH = N_NODES // OCH


def _sc_segment_sum(x, src2d, dst2d, zeros, cpw):
    mesh = plsc.VectorSubcoreMesh(core_axis_name="c", subcore_axis_name="s")

    assert cpw % 4 == 0 and cpw >= 8
    L = cpw // 2

    @functools.partial(
        pl.kernel,
        out_type=jax.ShapeDtypeStruct((NC * N_NODES, D), jnp.float32),
        mesh=mesh,
        scratch_types=[
            pltpu.VMEM_SHARED((N_NODES, D), jnp.float32),
            pltpu.VMEM((L, CH), jnp.int32),
            pltpu.VMEM((L, CH), jnp.int32),
            pltpu.VMEM((2, CH, D), jnp.float32),
            pltpu.SemaphoreType.DMA,
            pltpu.SemaphoreType.DMA,
        ],
    )
    def seg_sum(x_hbm, src_hbm, dst_hbm, z_hbm, out_hbm, acc, sidx, didx, bufs,
                g0, g1):
        gs = (g0, g1)
        c = lax.axis_index("c")
        s = lax.axis_index("s")
        wid = s * NC + c

        pltpu.sync_copy(z_hbm, bufs.at[0].at[pl.ds(0, OCH)])
        for k0 in range(-(-NCH // NS)):
            k = k0 * NS + s
            @pl.when(k < NCH)
            def _():
                pltpu.sync_copy(bufs.at[0].at[pl.ds(0, OCH)],
                                acc.at[pl.ds(k * OCH, OCH)])
        plsc.subcore_barrier()

        def start_gather(j, b):
            pltpu.async_copy(x_hbm.at[sidx.at[j]], bufs.at[b], gs[b])

        def wait_gather(j, b):
            pltpu.make_async_copy(x_hbm.at[sidx.at[j]], bufs.at[b], gs[b]).wait()

        for p in range(2):
            pltpu.sync_copy(src_hbm.at[pl.ds(wid * cpw + p * L, L)], sidx)
            pltpu.sync_copy(dst_hbm.at[pl.ds(wid * cpw + p * L, L)], didx)
            start_gather(0, 0)

            def slot(j, b):
                wait_gather(j, b)
                start_gather(lax.rem(j + 1, L), 1 - b)
                pltpu.sync_copy(bufs.at[b], acc.at[didx.at[j]], add=True)

            def round_body(g, carry):
                slot(2 * g, 0)
                slot(2 * g + 1, 1)
                return carry

            lax.fori_loop(0, L // 2, round_body, 0)
            wait_gather(0, 0)
        plsc.subcore_barrier()

        for k0 in range(-(-NCH // NS)):
            k = k0 * NS + s
            @pl.when(k < NCH)
            def _():
                pltpu.sync_copy(acc.at[pl.ds(k * OCH, OCH)],
                                bufs.at[0].at[pl.ds(0, OCH)])
                pltpu.sync_copy(bufs.at[0].at[pl.ds(0, OCH)],
                                out_hbm.at[pl.ds(c * N_NODES + k * OCH, OCH)])

    return seg_sum(x, src2d, dst2d, zeros)


def _tc_body(x_ref, p0_ref, p1_ref, wt_ref, b_ref, out_ref):
    m = p0_ref[...] + p1_ref[...]
    a = jnp.dot(x_ref[...], wt_ref[0:D, :], preferred_element_type=jnp.float32)
    a = a + jnp.dot(m, wt_ref[D:2 * D, :], preferred_element_type=jnp.float32)
    out_ref[...] = jnp.maximum(a + b_ref[...], 0.0)


def kernel(x, edge_index, W, b):
    n, d = x.shape
    e = edge_index.shape[1]
    assert n == N_NODES and d == D
    assert e % (NW * CH) == 0

    cpw = e // (NW * CH)
    src2d = edge_index[0].astype(jnp.int32).reshape(-1, CH)
    dst2d = edge_index[1].astype(jnp.int32).reshape(-1, CH)
    zeros = jnp.zeros((OCH, D), jnp.float32)

    partials = _sc_segment_sum(x, src2d, dst2d, zeros, cpw)
    p0 = partials[:N_NODES]
    p1 = partials[N_NODES:]

    wt = W.T
    b2 = b.reshape(1, D)
    br = 1000
    out = pl.pallas_call(
        _tc_body,
        grid=(N_NODES // br,),
        in_specs=[
            pl.BlockSpec((br, D), lambda i: (i, 0)),
            pl.BlockSpec((br, D), lambda i: (i, 0)),
            pl.BlockSpec((br, D), lambda i: (i, 0)),
            pl.BlockSpec((2 * D, D), lambda i: (0, 0)),
            pl.BlockSpec((1, D), lambda i: (0, 0)),
        ],
        out_specs=pl.BlockSpec((br, D), lambda i: (i, 0)),
        out_shape=jax.ShapeDtypeStruct((N_NODES, D), jnp.float32),
    )(x, p0, p1, wt, b2)
    return out


# speedup vs baseline: 10.6282x; 1.0350x over previous
import functools

import jax
import jax.numpy as jnp
from jax import lax
from jax.experimental import pallas as pl
from jax.experimental.pallas import tpu as pltpu
from jax.experimental.pallas import tpu_sc as plsc

N_NODES = 10000
D = 128
NC = 2
NS = 16
NW = NC * NS
CH = 125
OCH = 80
NCH = N_NODES // OCH


def _sc_segment_sum(x, edges3d, zeros, cpw):
    mesh = plsc.VectorSubcoreMesh(core_axis_name="c", subcore_axis_name="s")

    assert cpw % 4 == 0 and cpw >= 8
    L = cpw // 2

    @functools.partial(
        pl.kernel,
        out_type=jax.ShapeDtypeStruct((NC * N_NODES, D), jnp.float32),
        mesh=mesh,
        scratch_types=[
            pltpu.VMEM_SHARED((N_NODES, D), jnp.float32),
            pltpu.VMEM((L, CH), jnp.int32),
            pltpu.VMEM((L, CH), jnp.int32),
            pltpu.VMEM((2, CH, D), jnp.float32),
            pltpu.SemaphoreType.DMA,
            pltpu.SemaphoreType.DMA,
            pltpu.SemaphoreType.DMA,
        ],
    )
    def seg_sum(x_hbm, e_hbm, z_hbm, out_hbm, acc, sidx, didx, bufs, g0, g1, zs):
        gs = (g0, g1)
        c = lax.axis_index("c")
        s = lax.axis_index("s")
        wid = s * NC + c
        src_hbm = e_hbm.at[0]
        dst_hbm = e_hbm.at[1]

        for k0 in range(-(-NCH // NS)):
            k = k0 * NS + s
            @pl.when(k < NCH)
            def _():
                pltpu.async_copy(z_hbm, acc.at[pl.ds(k * OCH, OCH)], zs)
        pltpu.sync_copy(src_hbm.at[pl.ds(wid * cpw, L)], sidx)
        pltpu.sync_copy(dst_hbm.at[pl.ds(wid * cpw, L)], didx)
        for k0 in range(-(-NCH // NS)):
            k = k0 * NS + s
            @pl.when(k < NCH)
            def _():
                pltpu.make_async_copy(z_hbm, acc.at[pl.ds(k * OCH, OCH)], zs).wait()
        plsc.subcore_barrier()

        def start_gather(j, b):
            pltpu.async_copy(x_hbm.at[sidx.at[j]], bufs.at[b], gs[b])

        def wait_gather(j, b):
            pltpu.make_async_copy(x_hbm.at[sidx.at[j]], bufs.at[b], gs[b]).wait()

        for p in range(2):
            if p:
                pltpu.sync_copy(src_hbm.at[pl.ds(wid * cpw + p * L, L)], sidx)
                pltpu.sync_copy(dst_hbm.at[pl.ds(wid * cpw + p * L, L)], didx)
            start_gather(0, 0)

            def slot(j, b):
                wait_gather(j, b)
                start_gather(lax.rem(j + 1, L), 1 - b)
                pltpu.sync_copy(bufs.at[b], acc.at[didx.at[j]], add=True)

            def round_body(g, carry):
                slot(2 * g, 0)
                slot(2 * g + 1, 1)
                return carry

            lax.fori_loop(0, L // 2, round_body, 0)
            wait_gather(0, 0)
        plsc.subcore_barrier()

        for k0 in range(-(-NCH // NS)):
            k = k0 * NS + s
            @pl.when(k < NCH)
            def _():
                pltpu.async_copy(acc.at[pl.ds(k * OCH, OCH)],
                                 out_hbm.at[pl.ds(c * N_NODES + k * OCH, OCH)], zs)
        for k0 in range(-(-NCH // NS)):
            k = k0 * NS + s
            @pl.when(k < NCH)
            def _():
                pltpu.make_async_copy(acc.at[pl.ds(k * OCH, OCH)],
                                      out_hbm.at[pl.ds(c * N_NODES + k * OCH, OCH)],
                                      zs).wait()

    return seg_sum(x, edges3d, zeros)


def _tc_body(x_ref, p0_ref, p1_ref, w_ref, b_ref, out_ref):
    dn = (((1,), (1,)), ((), ()))
    m = p0_ref[...] + p1_ref[...]
    a = lax.dot_general(x_ref[...], w_ref[:, 0:D], dn,
                        preferred_element_type=jnp.float32)
    a = a + lax.dot_general(m, w_ref[:, D:2 * D], dn,
                            preferred_element_type=jnp.float32)
    out_ref[...] = jnp.maximum(a + b_ref[...], 0.0)


def kernel(x, edge_index, W, b):
    n, d = x.shape
    e = edge_index.shape[1]
    assert n == N_NODES and d == D
    assert e % (NW * CH) == 0

    cpw = e // (NW * CH)
    edges3d = edge_index.astype(jnp.int32).reshape(2, -1, CH)
    zeros = jnp.zeros((OCH, D), jnp.float32)

    partials = _sc_segment_sum(x, edges3d, zeros, cpw)

    b2 = b.reshape(1, D)
    br = 1000
    nb = N_NODES // br
    out = pl.pallas_call(
        _tc_body,
        grid=(nb,),
        in_specs=[
            pl.BlockSpec((br, D), lambda i: (i, 0)),
            pl.BlockSpec((br, D), lambda i: (i, 0)),
            pl.BlockSpec((br, D), lambda i: (i + nb, 0)),
            pl.BlockSpec((D, 2 * D), lambda i: (0, 0)),
            pl.BlockSpec((1, D), lambda i: (0, 0)),
        ],
        out_specs=pl.BlockSpec((br, D), lambda i: (i, 0)),
        out_shape=jax.ShapeDtypeStruct((N_NODES, D), jnp.float32),
    )(x, partials, partials, W, b2)
    return out


# speedup vs baseline: 12.7996x; 1.2043x over previous
import functools

import jax
import jax.numpy as jnp
from jax import lax
from jax.experimental import pallas as pl
from jax.experimental.pallas import tpu as pltpu
from jax.experimental.pallas import tpu_sc as plsc

N_NODES = 10000
D = 128
NC = 2
NS = 16
NW = NC * NS
CH = 128
OCH = 80
NCH = N_NODES // OCH


def _sc_segment_sum(x, edges, zeros, ncw, rem):
    mesh = plsc.VectorSubcoreMesh(core_axis_name="c", subcore_axis_name="s")
    assert ncw % 3 == 0 and ncw >= 6 and rem < NW

    @functools.partial(
        pl.kernel,
        out_type=jax.ShapeDtypeStruct((NC * N_NODES, D), jnp.float32),
        mesh=mesh,
        scratch_types=[
            pltpu.VMEM_SHARED((N_NODES, D), jnp.float32),
            pltpu.VMEM((3, CH), jnp.int32),
            pltpu.VMEM((3, CH), jnp.int32),
            pltpu.VMEM((3, CH, D), jnp.float32),
            pltpu.SemaphoreType.DMA,
            pltpu.SemaphoreType.DMA,
            pltpu.SemaphoreType.DMA,
            pltpu.SemaphoreType.DMA,
            pltpu.SemaphoreType.DMA,
            pltpu.SemaphoreType.DMA,
            pltpu.SemaphoreType.DMA,
        ],
    )
    def seg_sum(x_hbm, e_hbm, z_hbm, out_hbm, acc, sring, dring, bufs,
                g0, g1, g2, i0, i1, i2, zs):
        gs = (g0, g1, g2)
        iss = (i0, i1, i2)
        c = lax.axis_index("c")
        s = lax.axis_index("s")
        wid = s * NC + c
        base = wid * ncw

        for k0 in range(-(-NCH // NS)):
            k = k0 * NS + s
            @pl.when(k < NCH)
            def _():
                pltpu.async_copy(z_hbm, acc.at[pl.ds(k * OCH, OCH)], zs)
        for k0 in range(-(-NCH // NS)):
            k = k0 * NS + s
            @pl.when(k < NCH)
            def _():
                pltpu.make_async_copy(z_hbm, acc.at[pl.ds(k * OCH, OCH)], zs).wait()
        plsc.subcore_barrier()

        def chunk_off(j):
            return pl.multiple_of((base + lax.rem(j, ncw)) * CH, CH)

        def start_idx(j, t):
            off = chunk_off(j)
            pltpu.async_copy(e_hbm.at[0, pl.ds(off, CH)], sring.at[t], iss[t])
            pltpu.async_copy(e_hbm.at[1, pl.ds(off, CH)], dring.at[t], iss[t])

        def wait_idx(t):
            pltpu.make_async_copy(e_hbm.at[0, pl.ds(0, CH)], sring.at[t],
                                  iss[t]).wait()
            pltpu.make_async_copy(e_hbm.at[1, pl.ds(0, CH)], dring.at[t],
                                  iss[t]).wait()

        def start_gather(t):
            pltpu.async_copy(x_hbm.at[sring.at[t]], bufs.at[t], gs[t])

        def wait_gather(t):
            pltpu.make_async_copy(x_hbm.at[sring.at[t]], bufs.at[t], gs[t]).wait()

        start_idx(0, 0)
        start_idx(1, 1)
        start_idx(2, 2)
        wait_idx(0)
        start_gather(0)
        wait_idx(1)
        start_gather(1)

        def slot(j, t):
            wait_gather(t)
            pltpu.sync_copy(bufs.at[t], acc.at[dring.at[t]], add=True)
            start_idx(j + 3, t)
            wait_idx((t + 2) % 3)
            start_gather((t + 2) % 3)

        def round_body(r, carry):
            slot(3 * r, 0)
            slot(3 * r + 1, 1)
            slot(3 * r + 2, 2)
            return carry

        lax.fori_loop(0, ncw // 3, round_body, 0)
        wait_gather(0)
        wait_gather(1)
        wait_idx(2)

        if rem:
            @pl.when(wid < rem)
            def _():
                off = pl.multiple_of((NW * ncw + wid) * CH, CH)
                pltpu.sync_copy(e_hbm.at[0, pl.ds(off, CH)], sring.at[0])
                pltpu.sync_copy(e_hbm.at[1, pl.ds(off, CH)], dring.at[0])
                pltpu.async_copy(x_hbm.at[sring.at[0]], bufs.at[0], g0).wait()
                pltpu.sync_copy(bufs.at[0], acc.at[dring.at[0]], add=True)
        plsc.subcore_barrier()

        for k0 in range(-(-NCH // NS)):
            k = k0 * NS + s
            @pl.when(k < NCH)
            def _():
                pltpu.async_copy(acc.at[pl.ds(k * OCH, OCH)],
                                 out_hbm.at[pl.ds(c * N_NODES + k * OCH, OCH)], zs)
        for k0 in range(-(-NCH // NS)):
            k = k0 * NS + s
            @pl.when(k < NCH)
            def _():
                pltpu.make_async_copy(acc.at[pl.ds(k * OCH, OCH)],
                                      out_hbm.at[pl.ds(c * N_NODES + k * OCH, OCH)],
                                      zs).wait()

    return seg_sum(x, edges, zeros)


def _tc_body(x_ref, p0_ref, p1_ref, w_ref, b_ref, out_ref):
    dn = (((1,), (1,)), ((), ()))
    m = p0_ref[...] + p1_ref[...]
    a = lax.dot_general(x_ref[...], w_ref[:, 0:D], dn,
                        preferred_element_type=jnp.float32)
    a = a + lax.dot_general(m, w_ref[:, D:2 * D], dn,
                            preferred_element_type=jnp.float32)
    out_ref[...] = jnp.maximum(a + b_ref[...], 0.0)


def kernel(x, edge_index, W, b):
    n, d = x.shape
    e = edge_index.shape[1]
    assert n == N_NODES and d == D
    assert e % CH == 0

    nchk = e // CH
    ncw = nchk // NW - (nchk // NW) % 3
    rem = nchk - NW * ncw
    edges = edge_index.astype(jnp.int32)
    zeros = jnp.zeros((OCH, D), jnp.float32)

    partials = _sc_segment_sum(x, edges, zeros, ncw, rem)

    b2 = b.reshape(1, D)
    br = 1000
    nb = N_NODES // br
    out = pl.pallas_call(
        _tc_body,
        grid=(nb,),
        in_specs=[
            pl.BlockSpec((br, D), lambda i: (i, 0)),
            pl.BlockSpec((br, D), lambda i: (i, 0)),
            pl.BlockSpec((br, D), lambda i: (i + nb, 0)),
            pl.BlockSpec((D, 2 * D), lambda i: (0, 0)),
            pl.BlockSpec((1, D), lambda i: (0, 0)),
        ],
        out_specs=pl.BlockSpec((br, D), lambda i: (i, 0)),
        out_shape=jax.ShapeDtypeStruct((N_NODES, D), jnp.float32),
    )(x, partials, partials, W, b2)
    return out
